# Initial kernel scaffold; baseline (speedup 1.0000x reference)
import jax
import jax.numpy as jnp
from jax.experimental import pallas as pl


def kernel(queries, keys):
    raise NotImplementedError("write your pallas kernel here")


# speedup vs baseline: 1.7589x; 1.7589x over previous
import functools
import math

import jax
import jax.numpy as jnp
from jax import lax
from jax.experimental import pallas as pl
from jax.experimental.pallas import tpu as pltpu
from jax.experimental.pallas import tpu_sc as plsc

_K = 10
_BLK = 2048
_BIG = 2**30
_NEG = float("-inf")


def _topk_body(n_keys, q_ref, k_ref, idx_ref, w_ref, rv_ref, ri_ref):
    b = pl.program_id(0)
    nb = pl.num_programs(0)

    @pl.when(b == 0)
    def _init():
        rv_ref[...] = jnp.full(rv_ref.shape, _NEG, jnp.float32)
        ri_ref[...] = jnp.zeros(ri_ref.shape, jnp.int32)

    q = q_ref[...]
    k = k_ref[...]
    p = lax.dot_general(q, k, (((1,), (1,)), ((), ())),
                        preferred_element_type=jnp.float32)
    k2r = lax.dot_general(jnp.ones((1, 16), jnp.float32), k * k,
                          (((1,), (1,)), ((), ())),
                          precision=lax.Precision.HIGHEST,
                          preferred_element_type=jnp.float32)
    s = 2.0 * p - k2r
    col = lax.broadcasted_iota(jnp.int32, s.shape, 1)
    s = jnp.where(col + b * _BLK < n_keys, s, _NEG)

    bvals, bidx = [], []
    for _ in range(_K):
        m = jnp.max(s, axis=1, keepdims=True)
        am = jnp.min(jnp.where(s == m, col, _BIG), axis=1, keepdims=True)
        bvals.append(m)
        bidx.append(am + b * _BLK)
        s = jnp.where(col == am, _NEG, s)

    cv = jnp.concatenate([rv_ref[...]] + bvals, axis=1)
    ci = jnp.concatenate([ri_ref[...]] + bidx, axis=1)
    col2 = lax.broadcasted_iota(jnp.int32, cv.shape, 1)
    nv, ni = [], []
    for _ in range(_K):
        m = jnp.max(cv, axis=1, keepdims=True)
        am = jnp.min(jnp.where(cv == m, col2, _BIG), axis=1, keepdims=True)
        sel = col2 == am
        nv.append(m)
        ni.append(jnp.sum(jnp.where(sel, ci, 0), axis=1, keepdims=True))
        cv = jnp.where(sel, _NEG, cv)
    rv_ref[...] = jnp.concatenate(nv, axis=1)
    ri_ref[...] = jnp.concatenate(ni, axis=1)

    @pl.when(b == nb - 1)
    def _finish():
        v = jnp.concatenate(nv, axis=1)
        i = jnp.concatenate(ni, axis=1)
        e = jnp.exp(v - v[:, 0:1])
        w = e / jnp.sum(e, axis=1, keepdims=True)
        idx_ref[...] = jnp.concatenate([i] + [i[:, 0:1]] * 6, axis=1)
        w_ref[...] = jnp.concatenate(
            [jnp.broadcast_to(w[:, j:j + 1], (w.shape[0], 16))
             for j in range(_K)], axis=1)


def _tc_topk(queries, keys_padded, n_keys):
    nq = queries.shape[0]
    nb = keys_padded.shape[0] // _BLK
    return pl.pallas_call(
        functools.partial(_topk_body, n_keys),
        grid=(nb,),
        in_specs=[
            pl.BlockSpec((nq, 16), lambda b: (0, 0)),
            pl.BlockSpec((_BLK, 16), lambda b: (b, 0)),
        ],
        out_specs=[
            pl.BlockSpec((nq, 16), lambda b: (0, 0)),
            pl.BlockSpec((nq, 16 * _K), lambda b: (0, 0)),
        ],
        out_shape=[
            jax.ShapeDtypeStruct((nq, 16), jnp.int32),
            jax.ShapeDtypeStruct((nq, 16 * _K), jnp.float32),
        ],
        scratch_shapes=[
            pltpu.VMEM((nq, _K), jnp.float32),
            pltpu.VMEM((nq, _K), jnp.int32),
        ],
        compiler_params=pltpu.CompilerParams(
            dimension_semantics=("arbitrary",)),
    )(queries, keys_padded)


def _sc_aggregate(keys, idx_flat, w_rep):
    nq = w_rep.shape[0]
    info = plsc.get_sparse_core_info()
    nw = info.num_cores * info.num_subcores
    qpw = nq // nw
    mesh = plsc.VectorSubcoreMesh(core_axis_name="c", subcore_axis_name="s")

    @functools.partial(
        pl.kernel,
        mesh=mesh,
        out_type=jax.ShapeDtypeStruct((nq, 16), jnp.float32),
        scratch_types=[
            pltpu.VMEM((qpw * 16,), jnp.int32),
            pltpu.VMEM((qpw * 16, 16), jnp.float32),
            pltpu.VMEM((qpw, 16 * _K), jnp.float32),
            pltpu.VMEM((qpw, 16), jnp.float32),
            pltpu.SemaphoreType.DMA,
        ],
        compiler_params=pltpu.CompilerParams(use_tc_tiling_on_sc=False),
    )
    def body(keys_hbm, idx_hbm, w_hbm, out_hbm, idx_v, rows_v, w_v, out_v,
             sem):
        wid = lax.axis_index("s") * info.num_cores + lax.axis_index("c")
        qbase = wid * qpw
        pltpu.sync_copy(idx_hbm.at[pl.ds(qbase * 16, qpw * 16)], idx_v)
        pltpu.async_copy(keys_hbm.at[idx_v], rows_v, sem).wait()
        pltpu.sync_copy(w_hbm.at[pl.ds(qbase, qpw)], w_v)
        for q in range(qpw):
            acc = rows_v[q * 16] * w_v[q, pl.ds(0, 16)]
            for j in range(1, _K):
                acc = acc + rows_v[q * 16 + j] * w_v[q, pl.ds(j * 16, 16)]
            out_v[q] = acc
        pltpu.sync_copy(out_v, out_hbm.at[pl.ds(qbase, qpw)])

    return body(keys, idx_flat, w_rep)


def kernel(queries, keys):
    n_keys = keys.shape[0]
    nb = math.ceil(n_keys / _BLK)
    keys_padded = jnp.pad(keys, ((0, nb * _BLK - n_keys), (0, 0)))
    idx16, w_rep = _tc_topk(queries, keys_padded, n_keys)
    return _sc_aggregate(keys, idx16.reshape(-1), w_rep)


# speedup vs baseline: 2.5434x; 1.4460x over previous
import functools
import math

import jax
import jax.numpy as jnp
from jax import lax
from jax.experimental import pallas as pl
from jax.experimental.pallas import tpu as pltpu
from jax.experimental.pallas import tpu_sc as plsc

_K = 10
_BLK = 2048
_BIGF = 3.0e38
_NEG = float("-inf")


def _topk_body(n_keys, q_ref, k_ref, vals_ref, idxf_ref):
    b = pl.program_id(0)

    q = q_ref[...]
    k = k_ref[...]
    p = lax.dot_general(q, k, (((1,), (1,)), ((), ())),
                        preferred_element_type=jnp.float32)
    k2r = lax.dot_general(jnp.ones((1, 16), jnp.float32), k * k,
                          (((1,), (1,)), ((), ())),
                          precision=lax.Precision.HIGHEST,
                          preferred_element_type=jnp.float32)
    s = 2.0 * p - k2r
    colf = lax.broadcasted_iota(jnp.int32, s.shape, 1).astype(jnp.float32)
    lim = (n_keys - b * _BLK).astype(jnp.float32)
    s = jnp.where(colf < lim, s, _NEG)

    bvals, bidx = [], []
    for _ in range(_K):
        m = jnp.max(s, axis=1, keepdims=True)
        am = jnp.min(jnp.where(s == m, colf, _BIGF), axis=1, keepdims=True)
        bvals.append(m)
        bidx.append(am + (b * _BLK).astype(jnp.float32))
        s = jnp.where(colf == am, _NEG, s)

    vpad = [jnp.full((q.shape[0], 1), _NEG, jnp.float32)] * 6
    ipad = [jnp.zeros((q.shape[0], 1), jnp.float32)] * 6
    vals_ref[0] = jnp.concatenate(bvals + vpad, axis=1)
    idxf_ref[0] = jnp.concatenate(bidx + ipad, axis=1)


def _merge_body(cv_ref, ci_ref, idx_ref, w_ref):
    cv = cv_ref[...]
    ci = ci_ref[...]
    nv, ni = [], []
    for _ in range(_K):
        m = jnp.max(cv, axis=1, keepdims=True)
        ai = jnp.min(jnp.where(cv == m, ci, _BIGF), axis=1, keepdims=True)
        nv.append(m)
        ni.append(ai)
        cv = jnp.where(ci == ai, _NEG, cv)
    v = jnp.concatenate(nv, axis=1)
    i = jnp.concatenate(ni, axis=1).astype(jnp.int32)
    e = jnp.exp(v - v[:, 0:1])
    w = e / jnp.sum(e, axis=1, keepdims=True)
    idx_ref[...] = jnp.concatenate([i] + [i[:, 0:1]] * 6, axis=1)
    w_ref[...] = jnp.concatenate(
        [jnp.broadcast_to(w[:, j:j + 1], (w.shape[0], 16))
         for j in range(_K)], axis=1)


def _tc_topk(queries, keys_padded, n_keys):
    nq = queries.shape[0]
    nb = keys_padded.shape[0] // _BLK
    vals3, idxf3 = pl.pallas_call(
        functools.partial(_topk_body, n_keys),
        grid=(nb,),
        in_specs=[
            pl.BlockSpec((nq, 16), lambda b: (0, 0)),
            pl.BlockSpec((_BLK, 16), lambda b: (b, 0)),
        ],
        out_specs=[
            pl.BlockSpec((1, nq, 16), lambda b: (b, 0, 0)),
            pl.BlockSpec((1, nq, 16), lambda b: (b, 0, 0)),
        ],
        out_shape=[
            jax.ShapeDtypeStruct((nb, nq, 16), jnp.float32),
            jax.ShapeDtypeStruct((nb, nq, 16), jnp.float32),
        ],
        compiler_params=pltpu.CompilerParams(
            dimension_semantics=("arbitrary",)),
    )(queries, keys_padded)
    cv = vals3.transpose(1, 0, 2).reshape(nq, nb * 16)
    ci = idxf3.transpose(1, 0, 2).reshape(nq, nb * 16)
    return pl.pallas_call(
        _merge_body,
        out_shape=[
            jax.ShapeDtypeStruct((nq, 16), jnp.int32),
            jax.ShapeDtypeStruct((nq, 16 * _K), jnp.float32),
        ],
    )(cv, ci)


def _sc_aggregate(keys, idx_flat, w_rep):
    nq = w_rep.shape[0]
    info = plsc.get_sparse_core_info()
    nw = info.num_cores * info.num_subcores
    qpw = nq // nw
    mesh = plsc.VectorSubcoreMesh(core_axis_name="c", subcore_axis_name="s")

    @functools.partial(
        pl.kernel,
        mesh=mesh,
        out_type=jax.ShapeDtypeStruct((nq, 16), jnp.float32),
        scratch_types=[
            pltpu.VMEM((qpw * 16,), jnp.int32),
            pltpu.VMEM((qpw * 16, 16), jnp.float32),
            pltpu.VMEM((qpw, 16 * _K), jnp.float32),
            pltpu.VMEM((qpw, 16), jnp.float32),
            pltpu.SemaphoreType.DMA,
        ],
        compiler_params=pltpu.CompilerParams(use_tc_tiling_on_sc=False),
    )
    def body(keys_hbm, idx_hbm, w_hbm, out_hbm, idx_v, rows_v, w_v, out_v,
             sem):
        wid = lax.axis_index("s") * info.num_cores + lax.axis_index("c")
        qbase = wid * qpw
        pltpu.sync_copy(idx_hbm.at[pl.ds(qbase * 16, qpw * 16)], idx_v)
        pltpu.async_copy(keys_hbm.at[idx_v], rows_v, sem).wait()
        pltpu.sync_copy(w_hbm.at[pl.ds(qbase, qpw)], w_v)
        for q in range(qpw):
            acc = rows_v[q * 16] * w_v[q, pl.ds(0, 16)]
            for j in range(1, _K):
                acc = acc + rows_v[q * 16 + j] * w_v[q, pl.ds(j * 16, 16)]
            out_v[q] = acc
        pltpu.sync_copy(out_v, out_hbm.at[pl.ds(qbase, qpw)])

    return body(keys, idx_flat, w_rep)


def kernel(queries, keys):
    n_keys = keys.shape[0]
    nb = math.ceil(n_keys / _BLK)
    keys_padded = jnp.pad(keys, ((0, nb * _BLK - n_keys), (0, 0)))
    idx16, w_rep = _tc_topk(queries, keys_padded, n_keys)
    return _sc_aggregate(keys, idx16.reshape(-1), w_rep)
